# baseline (speedup 1.0000x reference)
import functools
import numpy as np
import jax
import jax.numpy as jnp
from jax import lax
from jax.experimental import pallas as pl
from jax.experimental.pallas import tpu as pltpu
from jax.experimental.pallas import tpu_sc as plsc

B = 8
KN = 3072
NIMP = 768
NCOV = 256
NPT = 1024
FC = 160
FS = 128
CS = 32
NC, NS, L = 2, 16, 16
NW = NC * NS


def _rotl(x, r):
    r = np.uint32(r)
    return (x << r) | (x >> np.uint32(32 - r))


def _tf2x32(k1, k2, x1, x2):
    rots = (np.uint32([13, 15, 26, 6]), np.uint32([17, 29, 16, 24]))
    k3 = np.uint32(k1 ^ k2 ^ np.uint32(0x1BD11BDA))
    x = [x1 + k1, x2 + k2]
    ks = [k2, k3, k1]
    for i in range(5):
        for r in rots[i % 2]:
            x[0] = x[0] + x[1]
            x[1] = _rotl(x[1], r)
            x[1] = x[0] ^ x[1]
        x = [x[0] + ks[0], x[1] + ks[1] + np.uint32(i + 1)]
        ks = ks[1:] + ks[:1]
    return x


def _np_uniform(key, shape):
    n = int(np.prod(shape))
    cnt = np.arange(n, dtype=np.uint64)
    hi = (cnt >> np.uint64(32)).astype(np.uint32)
    lo = (cnt & np.uint64(0xFFFFFFFF)).astype(np.uint32)
    b1, b2 = _tf2x32(key[0], key[1], hi, lo)
    bits = (b1 ^ b2).reshape(shape)
    fb = (bits >> np.uint32(9)) | np.uint32(0x3F800000)
    f = fb.view(np.float32) - np.float32(1.0)
    return np.maximum(np.float32(0.0), f * np.float32(1.0) + np.float32(0.0))


def _build_consts():
    key = np.uint32([0, 42])
    b1, b2 = _tf2x32(key[0], key[1], np.uint32([0, 0]), np.uint32([0, 1]))
    ks = np.stack([b1, b2], axis=1)
    over = _np_uniform(ks[0], (B, KN, 2))
    cov = _np_uniform(ks[1], (B, NCOV, 2))
    return over, cov


def _bilin_params_np(pts, S):
    gx = (2.0 * pts[..., 0] - 1.0).astype(np.float32)
    gy = (2.0 * pts[..., 1] - 1.0).astype(np.float32)
    ix = (((gx + 1.0) * S - 1.0) / 2.0).astype(np.float32)
    iy = (((gy + 1.0) * S - 1.0) / 2.0).astype(np.float32)
    x0 = np.floor(ix); x1 = x0 + 1.0
    y0 = np.floor(iy); y1 = y0 + 1.0
    wx1 = (ix - x0).astype(np.float32); wx0 = (1.0 - wx1).astype(np.float32)
    wy1 = (iy - y0).astype(np.float32); wy0 = (1.0 - wy1).astype(np.float32)
    idxs = []; wgts = []
    for (xi, yi, w) in ((x0, y0, wx0 * wy0), (x1, y0, wx1 * wy0),
                        (x0, y1, wx0 * wy1), (x1, y1, wx1 * wy1)):
        valid = (xi >= 0) & (xi <= S - 1) & (yi >= 0) & (yi <= S - 1)
        xc = np.clip(xi, 0, S - 1).astype(np.int32)
        yc = np.clip(yi, 0, S - 1).astype(np.int32)
        idxs.append(yc * S + xc)
        wgts.append((w * valid.astype(np.float32)).astype(np.float32))
    return np.stack(idxs, axis=0), np.stack(wgts, axis=0)


_OVER, _COV = _build_consts()
_OI, _OW = _bilin_params_np(_OVER, CS)
_OIDX_T = np.ascontiguousarray(np.transpose(_OI, (1, 0, 2)))
_OW_T = np.ascontiguousarray(np.transpose(_OW, (1, 0, 2)))
_OVER_T = np.ascontiguousarray(np.transpose(_OVER, (0, 2, 1)))
_COV_T = np.ascontiguousarray(np.transpose(_COV, (0, 2, 1)))

_mesh = plsc.VectorSubcoreMesh(core_axis_name="c", subcore_axis_name="s",
                               num_cores=NC, num_subcores=NS)


def _wid():
    return lax.axis_index("s") * NC + lax.axis_index("c")


def _unc_body(outf_hbm, oidx_hbm, ow_hbm, unc_hbm,
              maps_v, mx_v, md_v, idx_v, w_v, u_v, sem):
    wid = _wid()
    b = wid // 4
    q = wid % 4
    qs = q * (KN // 4)
    hs = [pltpu.async_copy(outf_hbm.at[b], maps_v, sem)]
    for j in range(4):
        hs.append(pltpu.async_copy(oidx_hbm.at[b, j, pl.ds(qs, KN // 4)],
                                   idx_v.at[j], sem))
        hs.append(pltpu.async_copy(ow_hbm.at[b, j, pl.ds(qs, KN // 4)],
                                   w_v.at[j], sem))
    for h in hs:
        h.wait()

    def mkmaps(g, _):
        a = maps_v[pl.ds(g * L, L)]
        bb = maps_v[pl.ds(1024 + g * L, L)]
        c = maps_v[pl.ds(2048 + g * L, L)]
        mx = jnp.maximum(jnp.maximum(a, bb), c)
        md = jnp.maximum(jnp.minimum(a, bb), jnp.minimum(jnp.maximum(a, bb), c))
        mx_v[pl.ds(g * L, L)] = mx
        md_v[pl.ds(g * L, L)] = md
        return _
    lax.fori_loop(0, 1024 // L, mkmaps, None)

    def grp(g, _):
        acc0 = None; acc1 = None
        for j in range(4):
            vi = idx_v[j, pl.ds(g * L, L)]
            vw = w_v[j, pl.ds(g * L, L)]
            g0 = plsc.load_gather(mx_v, [vi]) * vw
            g1 = plsc.load_gather(md_v, [vi]) * vw
            acc0 = g0 if acc0 is None else acc0 + g0
            acc1 = g1 if acc1 is None else acc1 + g1
        u_v[pl.ds(g * L, L)] = -(acc0 - acc1)
        return _
    lax.fori_loop(0, (KN // 4) // L, grp, None)
    pltpu.sync_copy(u_v, unc_hbm.at[b, pl.ds(qs, KN // 4)])


def _k_unc(outf, oidx, ow):
    return pl.kernel(
        _unc_body,
        out_type=jax.ShapeDtypeStruct((B, KN), jnp.float32),
        mesh=_mesh,
        compiler_params=pltpu.CompilerParams(needs_layout_passes=False),
        scratch_types=[
            pltpu.VMEM((3 * 1024,), jnp.float32),
            pltpu.VMEM((1024,), jnp.float32),
            pltpu.VMEM((1024,), jnp.float32),
            pltpu.VMEM((4, KN // 4), jnp.int32),
            pltpu.VMEM((4, KN // 4), jnp.float32),
            pltpu.VMEM((KN // 4,), jnp.float32),
            pltpu.SemaphoreType.DMA,
        ],
    )(outf, oidx, ow)


_RB = 768


def _rank_body(u_ref, ut_ref, rank_ref):
    un_all = ut_ref[0]
    um_all = u_ref[0]
    mids = lax.broadcasted_iota(jnp.int32, (_RB, _RB), 1)
    nids = lax.broadcasted_iota(jnp.int32, (_RB, _RB), 0)
    ones = jnp.ones((KN, 1), jnp.float32)
    for nb in range(KN // _RB):
        un = un_all[nb * _RB:(nb + 1) * _RB]
        preds = []
        for mb in range(KN // _RB):
            um = um_all[:, mb * _RB:(mb + 1) * _RB]
            if mb == nb:
                pred = (um > un) | ((um == un) & (mids < nids))
            elif mb < nb:
                pred = um >= un
            else:
                pred = um > un
            preds.append(pred.astype(jnp.float32))
        big = jnp.concatenate(preds, axis=1)
        cnt = jnp.dot(big, ones, preferred_element_type=jnp.float32)
        rank_ref[0, 0, nb * _RB:(nb + 1) * _RB] = cnt[:, 0].astype(jnp.int32)


def _k_rank(unc, unc_t):
    out = pl.pallas_call(
        _rank_body,
        grid=(B,),
        in_specs=[
            pl.BlockSpec((1, 1, KN), lambda b: (b, 0, 0)),
            pl.BlockSpec((1, KN, 1), lambda b: (b, 0, 0)),
        ],
        out_specs=pl.BlockSpec((1, 1, KN), lambda b: (b, 0, 0)),
        out_shape=jax.ShapeDtypeStruct((B, 1, KN), jnp.int32),
    )(unc.reshape(B, 1, KN), unc_t)
    return out.reshape(B, KN)


def _params16(px, py, S):
    ix = px * np.float32(S) - np.float32(0.5)
    iy = py * np.float32(S) - np.float32(0.5)
    tx = ix.astype(jnp.int32)
    x0i = tx - jnp.where(ix < tx.astype(jnp.float32), 1, 0)
    ty = iy.astype(jnp.int32)
    y0i = ty - jnp.where(iy < ty.astype(jnp.float32), 1, 0)
    wx1 = ix - x0i.astype(jnp.float32); wx0 = np.float32(1.0) - wx1
    wy1 = iy - y0i.astype(jnp.float32); wy0 = np.float32(1.0) - wy1
    res = []
    for (xi, yi, w) in ((x0i, y0i, wx0 * wy0), (x0i + 1, y0i, wx1 * wy0),
                        (x0i, y0i + 1, wx0 * wy1), (x0i + 1, y0i + 1, wx1 * wy1)):
        valid = (xi >= 0) & (xi <= S - 1) & (yi >= 0) & (yi <= S - 1)
        xc = jnp.clip(xi, 0, S - 1)
        yc = jnp.clip(yi, 0, S - 1)
        res.append((yc * S + xc, jnp.where(valid, w, np.float32(0.0))))
    return res


_QI = NIMP // 4
_QC = NCOV // 4


def _sel_body(rank_hbm, outf_hbm, overt_hbm, covt_hbm,
              pts_hbm, gidx_hbm, gw_hbm, cn_hbm,
              r_v, ox_v, oy_v, cx_v, cy_v, map_v, sel_v,
              pts_v, gidx_v, gw_v, cn_v, sem):
    wid = _wid()
    b = wid // 4
    q = wid % 4
    iota = lax.iota(jnp.int32, L)
    hs = [pltpu.async_copy(rank_hbm.at[b], r_v, sem),
          pltpu.async_copy(overt_hbm.at[b, 0], ox_v, sem),
          pltpu.async_copy(overt_hbm.at[b, 1], oy_v, sem),
          pltpu.async_copy(covt_hbm.at[b, 0, pl.ds(q * _QC, _QC)], cx_v, sem),
          pltpu.async_copy(covt_hbm.at[b, 1, pl.ds(q * _QC, _QC)], cy_v, sem),
          pltpu.async_copy(outf_hbm.at[b], map_v, sem)]
    for h in hs:
        h.wait()
    lo = q * _QI

    def scat(g, _):
        r = r_v[pl.ds(g * L, L)] - lo
        nvec = iota + g * L
        plsc.store_scatter(sel_v, [jnp.clip(r, 0, _QI - 1)], nvec,
                           mask=(r >= 0) & (r < _QI))
        return _
    lax.fori_loop(0, KN // L, scat, None)

    def process(nbase, px, py):
        pidx = iota + nbase
        plsc.store_scatter(pts_v, [pidx * 2], px)
        plsc.store_scatter(pts_v, [pidx * 2 + 1], py)
        cprm = _params16(px, py, CS)
        for ch in range(3):
            acc = None
            for (lin, w) in cprm:
                g = plsc.load_gather(map_v, [lin + ch * 1024]) * w
                acc = g if acc is None else acc + g
            plsc.store_scatter(cn_v, [pidx * 4 + ch], acc)
        plsc.store_scatter(cn_v, [pidx * 4 + 3], jnp.zeros((L,), jnp.float32))
        fprm = _params16(px, py, FS)
        for j, (lin, w) in enumerate(fprm):
            plsc.store_scatter(gidx_v, [pidx * 4 + j], lin + b * (FS * FS))
            plsc.store_scatter(gw_v, [pidx * 4 + j], w)

    def grp_imp(g, _):
        s = sel_v[pl.ds(g * L, L)]
        px = plsc.load_gather(ox_v, [s])
        py = plsc.load_gather(oy_v, [s])
        process(lo + g * L, px, py)
        return _
    lax.fori_loop(0, _QI // L, grp_imp, None)

    def grp_cov(g, _):
        px = cx_v[pl.ds(g * L, L)]
        py = cy_v[pl.ds(g * L, L)]
        process(NIMP + q * _QC + g * L, px, py)
        return _
    lax.fori_loop(0, _QC // L, grp_cov, None)

    ho = [pltpu.async_copy(pts_v.at[pl.ds(lo * 2, _QI * 2)],
                           pts_hbm.at[b, pl.ds(lo * 2, _QI * 2)], sem),
          pltpu.async_copy(pts_v.at[pl.ds(2 * NIMP + q * _QC * 2, _QC * 2)],
                           pts_hbm.at[b, pl.ds(2 * NIMP + q * _QC * 2, _QC * 2)], sem)]
    for (src, dst) in ((gidx_v, gidx_hbm), (gw_v, gw_hbm), (cn_v, cn_hbm)):
        ho.append(pltpu.async_copy(src.at[pl.ds(lo * 4, _QI * 4)],
                                   dst.at[b, pl.ds(lo * 4, _QI * 4)], sem))
        ho.append(pltpu.async_copy(src.at[pl.ds(4 * NIMP + q * _QC * 4, _QC * 4)],
                                   dst.at[b, pl.ds(4 * NIMP + q * _QC * 4, _QC * 4)], sem))
    for h in ho:
        h.wait()


def _k_sel(rank, outf, overt, covt):
    return pl.kernel(
        _sel_body,
        out_type=(
            jax.ShapeDtypeStruct((B, 2 * NPT), jnp.float32),
            jax.ShapeDtypeStruct((B, 4 * NPT), jnp.int32),
            jax.ShapeDtypeStruct((B, 4 * NPT), jnp.float32),
            jax.ShapeDtypeStruct((B, 4 * NPT), jnp.float32),
        ),
        mesh=_mesh,
        compiler_params=pltpu.CompilerParams(needs_layout_passes=False),
        scratch_types=[
            pltpu.VMEM((KN,), jnp.int32),
            pltpu.VMEM((KN,), jnp.float32),
            pltpu.VMEM((KN,), jnp.float32),
            pltpu.VMEM((_QC,), jnp.float32),
            pltpu.VMEM((_QC,), jnp.float32),
            pltpu.VMEM((3 * 1024,), jnp.float32),
            pltpu.VMEM((_QI,), jnp.int32),
            pltpu.VMEM((2 * NPT,), jnp.float32),
            pltpu.VMEM((4 * NPT,), jnp.int32),
            pltpu.VMEM((4 * NPT,), jnp.float32),
            pltpu.VMEM((4 * NPT,), jnp.float32),
            pltpu.SemaphoreType.DMA,
        ],
    )(rank, outf, overt, covt)


_PB = 1024


def _proj_body(res_ref, w_ref, proj_ref):
    r = res_ref[0]
    w = w_ref[...]
    h = lax.dot_general(w, r, (((1,), (0,)), ((), ())),
                        precision=lax.Precision.HIGHEST,
                        preferred_element_type=jnp.float32)
    proj_ref[0] = h.T


def _k_proj(res, w1ft):
    return pl.pallas_call(
        _proj_body,
        grid=(B, (FS * FS) // _PB),
        in_specs=[
            pl.BlockSpec((1, FC, _PB), lambda b, pb: (b, 0, pb)),
            pl.BlockSpec((128, FC), lambda b, pb: (0, 0)),
        ],
        out_specs=pl.BlockSpec((1, _PB, 128), lambda b, pb: (b, pb, 0)),
        out_shape=jax.ShapeDtypeStruct((B, FS * FS, 128), jnp.float32),
    )(res, w1ft)


_CH = 128
_CPT = NPT // 4


def _fine_body(res2t_hbm, gidx_hbm, gw_hbm, fine_hbm,
               idx0, idx1, w0, w1, rows0, rows1, out_v, sem0, sem1):
    wid = _wid()
    b = wid // 4
    q = wid % 4
    base = q * (4 * _CPT)
    nchunks = (4 * _CPT) // _CH
    ppc = _CH // 4
    idxs = (idx0, idx1); wv = (w0, w1); rows = (rows0, rows1); sems = (sem0, sem1)

    def start(k):
        s = k % 2
        pltpu.sync_copy(gidx_hbm.at[b, pl.ds(base + k * _CH, _CH)], idxs[s])
        pltpu.sync_copy(gw_hbm.at[b, pl.ds(base + k * _CH, _CH)], wv[s])
        return pltpu.async_copy(res2t_hbm.at[idxs[s]], rows[s], sems[s])

    handles = [start(0), None]
    for k in range(nchunks):
        s = k % 2
        handles[s].wait()
        if k + 1 < nchunks:
            handles[1 - s] = start(k + 1)
        rows_v = rows[s]
        w_v = wv[s]

        def point(p, _):
            rb = p * 4
            ws = [plsc.load_gather(w_v, [jnp.full((L,), rb + j, jnp.int32)])
                  for j in range(4)]
            for c in range(128 // L):
                acc = None
                for j in range(4):
                    t = ws[j] * rows_v[rb + j, pl.ds(c * L, L)]
                    acc = t if acc is None else acc + t
                out_v[p, pl.ds(c * L, L)] = acc
            return _
        lax.fori_loop(0, ppc, point, None)
        pltpu.sync_copy(out_v, fine_hbm.at[b, pl.ds(q * _CPT + k * ppc, ppc), :])


def _k_fine(res2t, gidx, gw):
    return pl.kernel(
        _fine_body,
        out_type=jax.ShapeDtypeStruct((B, NPT, 128), jnp.float32),
        mesh=_mesh,
        compiler_params=pltpu.CompilerParams(needs_layout_passes=False),
        scratch_types=[
            pltpu.VMEM((_CH,), jnp.int32),
            pltpu.VMEM((_CH,), jnp.int32),
            pltpu.VMEM((_CH,), jnp.float32),
            pltpu.VMEM((_CH,), jnp.float32),
            pltpu.VMEM((_CH, 128), jnp.float32),
            pltpu.VMEM((_CH, 128), jnp.float32),
            pltpu.VMEM((_CH // 4, 128), jnp.float32),
            pltpu.SemaphoreType.DMA,
            pltpu.SemaphoreType.DMA,
        ],
    )(res2t, gidx, gw)


def _mlp_body(fine_ref, cn_ref, w1c_ref, b1_ref, w2_ref, b2_ref,
              w3_ref, b3_ref, rend_ref):
    h1 = fine_ref[0]
    cn = cn_ref[0]
    h1 = h1 + jnp.dot(cn, w1c_ref[...], preferred_element_type=jnp.float32)
    h1 = jnp.maximum(h1 + b1_ref[...], 0.0)
    h2 = jnp.dot(h1, w2_ref[...], preferred_element_type=jnp.float32)
    h2 = jnp.maximum(h2 + b2_ref[...], 0.0)
    r = lax.dot_general(w3_ref[...], h2, (((1,), (1,)), ((), ())),
                        preferred_element_type=jnp.float32)
    rend_ref[0] = r + b3_ref[...]


def _k_mlp(fine, cn, w1c, b1, w2, b2, w3, b3):
    return pl.pallas_call(
        _mlp_body,
        grid=(B,),
        in_specs=[
            pl.BlockSpec((1, NPT, 128), lambda b: (b, 0, 0)),
            pl.BlockSpec((1, NPT, 4), lambda b: (b, 0, 0)),
            pl.BlockSpec((4, 128), lambda b: (0, 0)),
            pl.BlockSpec((1, 128), lambda b: (0, 0)),
            pl.BlockSpec((128, 128), lambda b: (0, 0)),
            pl.BlockSpec((1, 128), lambda b: (0, 0)),
            pl.BlockSpec((3, 128), lambda b: (0, 0)),
            pl.BlockSpec((3, 1), lambda b: (0, 0)),
        ],
        out_specs=pl.BlockSpec((1, 3, NPT), lambda b: (b, 0, 0)),
        out_shape=jax.ShapeDtypeStruct((B, 3, NPT), jnp.float32),
    )(fine, cn, w1c, b1, w2, b2, w3, b3)


def kernel(x, res2, out, W1, b1, W2, b2, W3, b3):
    outf = out.reshape(B, 3 * 1024)
    unc = _k_unc(outf, jnp.asarray(_OIDX_T), jnp.asarray(_OW_T))
    rank = _k_rank(unc, unc.reshape(B, KN, 1))
    pts_f, gidx, gw, cn = _k_sel(rank, outf,
                                 jnp.asarray(_OVER_T), jnp.asarray(_COV_T))
    proj = _k_proj(res2.reshape(B, FC, FS * FS), W1[:, :FC])
    fine = _k_fine(proj.reshape(B * FS * FS, 128), gidx, gw)
    w1c = jnp.pad(W1[:, FC:], ((0, 0), (0, 1))).T
    rend = _k_mlp(fine, cn.reshape(B, NPT, 4), w1c, b1[None, :],
                  W2.T, b2[None, :], W3, b3[:, None])
    points = pts_f.reshape(B, NPT, 2)
    return (out, rend, points)


# speedup vs baseline: 1.3906x; 1.3906x over previous
import functools
import numpy as np
import jax
import jax.numpy as jnp
from jax import lax
from jax.experimental import pallas as pl
from jax.experimental.pallas import tpu as pltpu
from jax.experimental.pallas import tpu_sc as plsc

B = 8
KN = 3072
NIMP = 768
NCOV = 256
NPT = 1024
FC = 160
FS = 128
CS = 32
NC, NS, L = 2, 16, 16
NW = NC * NS


def _rotl(x, r):
    r = np.uint32(r)
    return (x << r) | (x >> np.uint32(32 - r))


def _tf2x32(k1, k2, x1, x2):
    rots = (np.uint32([13, 15, 26, 6]), np.uint32([17, 29, 16, 24]))
    k3 = np.uint32(k1 ^ k2 ^ np.uint32(0x1BD11BDA))
    x = [x1 + k1, x2 + k2]
    ks = [k2, k3, k1]
    for i in range(5):
        for r in rots[i % 2]:
            x[0] = x[0] + x[1]
            x[1] = _rotl(x[1], r)
            x[1] = x[0] ^ x[1]
        x = [x[0] + ks[0], x[1] + ks[1] + np.uint32(i + 1)]
        ks = ks[1:] + ks[:1]
    return x


def _np_uniform(key, shape):
    n = int(np.prod(shape))
    cnt = np.arange(n, dtype=np.uint64)
    hi = (cnt >> np.uint64(32)).astype(np.uint32)
    lo = (cnt & np.uint64(0xFFFFFFFF)).astype(np.uint32)
    b1, b2 = _tf2x32(key[0], key[1], hi, lo)
    bits = (b1 ^ b2).reshape(shape)
    fb = (bits >> np.uint32(9)) | np.uint32(0x3F800000)
    f = fb.view(np.float32) - np.float32(1.0)
    return np.maximum(np.float32(0.0), f * np.float32(1.0) + np.float32(0.0))


def _build_consts():
    key = np.uint32([0, 42])
    b1, b2 = _tf2x32(key[0], key[1], np.uint32([0, 0]), np.uint32([0, 1]))
    ks = np.stack([b1, b2], axis=1)
    over = _np_uniform(ks[0], (B, KN, 2))
    cov = _np_uniform(ks[1], (B, NCOV, 2))
    return over, cov


def _bilin_params_np(pts, S):
    gx = (2.0 * pts[..., 0] - 1.0).astype(np.float32)
    gy = (2.0 * pts[..., 1] - 1.0).astype(np.float32)
    ix = (((gx + 1.0) * S - 1.0) / 2.0).astype(np.float32)
    iy = (((gy + 1.0) * S - 1.0) / 2.0).astype(np.float32)
    x0 = np.floor(ix); x1 = x0 + 1.0
    y0 = np.floor(iy); y1 = y0 + 1.0
    wx1 = (ix - x0).astype(np.float32); wx0 = (1.0 - wx1).astype(np.float32)
    wy1 = (iy - y0).astype(np.float32); wy0 = (1.0 - wy1).astype(np.float32)
    idxs = []; wgts = []
    for (xi, yi, w) in ((x0, y0, wx0 * wy0), (x1, y0, wx1 * wy0),
                        (x0, y1, wx0 * wy1), (x1, y1, wx1 * wy1)):
        valid = (xi >= 0) & (xi <= S - 1) & (yi >= 0) & (yi <= S - 1)
        xc = np.clip(xi, 0, S - 1).astype(np.int32)
        yc = np.clip(yi, 0, S - 1).astype(np.int32)
        idxs.append(yc * S + xc)
        wgts.append((w * valid.astype(np.float32)).astype(np.float32))
    return np.stack(idxs, axis=0), np.stack(wgts, axis=0)


_OVER, _COV = _build_consts()
_OI, _OW = _bilin_params_np(_OVER, CS)
_OIDX_T = np.ascontiguousarray(np.transpose(_OI, (1, 0, 2)))
_OW_T = np.ascontiguousarray(np.transpose(_OW, (1, 0, 2)))
_OVER_T = np.ascontiguousarray(np.transpose(_OVER, (0, 2, 1)))
_COV_T = np.ascontiguousarray(np.transpose(_COV, (0, 2, 1)))

_mesh = plsc.VectorSubcoreMesh(core_axis_name="c", subcore_axis_name="s",
                               num_cores=NC, num_subcores=NS)


def _wid():
    return lax.axis_index("s") * NC + lax.axis_index("c")


def _unc_body(outf_hbm, oidx_hbm, ow_hbm, unc_hbm,
              maps_v, mx_v, md_v, idx_v, w_v, u_v, sem):
    wid = _wid()
    b = wid // 4
    q = wid % 4
    qs = q * (KN // 4)
    hs = [pltpu.async_copy(outf_hbm.at[b], maps_v, sem)]
    for j in range(4):
        hs.append(pltpu.async_copy(oidx_hbm.at[b, j, pl.ds(qs, KN // 4)],
                                   idx_v.at[j], sem))
        hs.append(pltpu.async_copy(ow_hbm.at[b, j, pl.ds(qs, KN // 4)],
                                   w_v.at[j], sem))
    for h in hs:
        h.wait()

    def mkmaps(g, _):
        a = maps_v[pl.ds(g * L, L)]
        bb = maps_v[pl.ds(1024 + g * L, L)]
        c = maps_v[pl.ds(2048 + g * L, L)]
        mx = jnp.maximum(jnp.maximum(a, bb), c)
        md = jnp.maximum(jnp.minimum(a, bb), jnp.minimum(jnp.maximum(a, bb), c))
        mx_v[pl.ds(g * L, L)] = mx
        md_v[pl.ds(g * L, L)] = md
        return _
    lax.fori_loop(0, 1024 // L, mkmaps, None)

    def grp(g, _):
        acc0 = None; acc1 = None
        for j in range(4):
            vi = idx_v[j, pl.ds(g * L, L)]
            vw = w_v[j, pl.ds(g * L, L)]
            g0 = plsc.load_gather(mx_v, [vi]) * vw
            g1 = plsc.load_gather(md_v, [vi]) * vw
            acc0 = g0 if acc0 is None else acc0 + g0
            acc1 = g1 if acc1 is None else acc1 + g1
        u_v[pl.ds(g * L, L)] = -(acc0 - acc1)
        return _
    lax.fori_loop(0, (KN // 4) // L, grp, None)
    pltpu.sync_copy(u_v, unc_hbm.at[b, pl.ds(qs, KN // 4)])


def _k_unc(outf, oidx, ow):
    return pl.kernel(
        _unc_body,
        out_type=jax.ShapeDtypeStruct((B, KN), jnp.float32),
        mesh=_mesh,
        compiler_params=pltpu.CompilerParams(needs_layout_passes=False),
        scratch_types=[
            pltpu.VMEM((3 * 1024,), jnp.float32),
            pltpu.VMEM((1024,), jnp.float32),
            pltpu.VMEM((1024,), jnp.float32),
            pltpu.VMEM((4, KN // 4), jnp.int32),
            pltpu.VMEM((4, KN // 4), jnp.float32),
            pltpu.VMEM((KN // 4,), jnp.float32),
            pltpu.SemaphoreType.DMA,
        ],
    )(outf, oidx, ow)


_RB = 768


def _rank_body(u_ref, ut_ref, rank_ref):
    un_all = ut_ref[0]
    um_all = u_ref[0]
    mids = lax.broadcasted_iota(jnp.int32, (_RB, _RB), 1)
    nids = lax.broadcasted_iota(jnp.int32, (_RB, _RB), 0)
    ones = jnp.ones((KN, 1), jnp.float32)
    for nb in range(KN // _RB):
        un = un_all[nb * _RB:(nb + 1) * _RB]
        preds = []
        for mb in range(KN // _RB):
            um = um_all[:, mb * _RB:(mb + 1) * _RB]
            if mb == nb:
                pred = (um > un) | ((um == un) & (mids < nids))
            elif mb < nb:
                pred = um >= un
            else:
                pred = um > un
            preds.append(pred.astype(jnp.float32))
        big = jnp.concatenate(preds, axis=1)
        cnt = jnp.dot(big, ones, preferred_element_type=jnp.float32)
        rank_ref[0, 0, nb * _RB:(nb + 1) * _RB] = cnt[:, 0].astype(jnp.int32)


def _k_rank(unc, unc_t):
    out = pl.pallas_call(
        _rank_body,
        grid=(B,),
        in_specs=[
            pl.BlockSpec((1, 1, KN), lambda b: (b, 0, 0)),
            pl.BlockSpec((1, KN, 1), lambda b: (b, 0, 0)),
        ],
        out_specs=pl.BlockSpec((1, 1, KN), lambda b: (b, 0, 0)),
        out_shape=jax.ShapeDtypeStruct((B, 1, KN), jnp.int32),
    )(unc.reshape(B, 1, KN), unc_t)
    return out.reshape(B, KN)


def _params16(px, py, S):
    ix = px * np.float32(S) - np.float32(0.5)
    iy = py * np.float32(S) - np.float32(0.5)
    tx = ix.astype(jnp.int32)
    x0i = tx - jnp.where(ix < tx.astype(jnp.float32), 1, 0)
    ty = iy.astype(jnp.int32)
    y0i = ty - jnp.where(iy < ty.astype(jnp.float32), 1, 0)
    wx1 = ix - x0i.astype(jnp.float32); wx0 = np.float32(1.0) - wx1
    wy1 = iy - y0i.astype(jnp.float32); wy0 = np.float32(1.0) - wy1
    res = []
    for (xi, yi, w) in ((x0i, y0i, wx0 * wy0), (x0i + 1, y0i, wx1 * wy0),
                        (x0i, y0i + 1, wx0 * wy1), (x0i + 1, y0i + 1, wx1 * wy1)):
        valid = (xi >= 0) & (xi <= S - 1) & (yi >= 0) & (yi <= S - 1)
        xc = jnp.clip(xi, 0, S - 1)
        yc = jnp.clip(yi, 0, S - 1)
        res.append((yc * S + xc, jnp.where(valid, w, np.float32(0.0))))
    return res


_QI = NIMP // 4
_QC = NCOV // 4


def _sel_body(rank_hbm, outf_hbm, overt_hbm, covt_hbm,
              pts_hbm, gidx_hbm, gw_hbm, cn_hbm,
              r_v, ox_v, oy_v, cx_v, cy_v, map_v, sel_v,
              pts_v, gidx_v, gw_v, cn_v, sem):
    wid = _wid()
    b = wid // 4
    q = wid % 4
    iota = lax.iota(jnp.int32, L)
    hs = [pltpu.async_copy(rank_hbm.at[b], r_v, sem),
          pltpu.async_copy(overt_hbm.at[b, 0], ox_v, sem),
          pltpu.async_copy(overt_hbm.at[b, 1], oy_v, sem),
          pltpu.async_copy(covt_hbm.at[b, 0, pl.ds(q * _QC, _QC)], cx_v, sem),
          pltpu.async_copy(covt_hbm.at[b, 1, pl.ds(q * _QC, _QC)], cy_v, sem),
          pltpu.async_copy(outf_hbm.at[b], map_v, sem)]
    for h in hs:
        h.wait()
    lo = q * _QI

    def scat(g, _):
        r = r_v[pl.ds(g * L, L)] - lo
        nvec = iota + g * L
        plsc.store_scatter(sel_v, [jnp.clip(r, 0, _QI - 1)], nvec,
                           mask=(r >= 0) & (r < _QI))
        return _
    lax.fori_loop(0, KN // L, scat, None)

    def process(nbase, px, py):
        pidx = iota + nbase
        plsc.store_scatter(pts_v, [pidx * 2], px)
        plsc.store_scatter(pts_v, [pidx * 2 + 1], py)
        cprm = _params16(px, py, CS)
        for ch in range(3):
            acc = None
            for (lin, w) in cprm:
                g = plsc.load_gather(map_v, [lin + ch * 1024]) * w
                acc = g if acc is None else acc + g
            plsc.store_scatter(cn_v, [pidx * 4 + ch], acc)
        plsc.store_scatter(cn_v, [pidx * 4 + 3], jnp.zeros((L,), jnp.float32))
        fprm = _params16(px, py, FS)
        for j, (lin, w) in enumerate(fprm):
            plsc.store_scatter(gidx_v, [pidx * 4 + j], lin + b * (FS * FS))
            plsc.store_scatter(gw_v, [pidx * 4 + j], w)

    def grp_imp(g, _):
        s = sel_v[pl.ds(g * L, L)]
        px = plsc.load_gather(ox_v, [s])
        py = plsc.load_gather(oy_v, [s])
        process(lo + g * L, px, py)
        return _
    lax.fori_loop(0, _QI // L, grp_imp, None)

    def grp_cov(g, _):
        px = cx_v[pl.ds(g * L, L)]
        py = cy_v[pl.ds(g * L, L)]
        process(NIMP + q * _QC + g * L, px, py)
        return _
    lax.fori_loop(0, _QC // L, grp_cov, None)

    ho = [pltpu.async_copy(pts_v.at[pl.ds(lo * 2, _QI * 2)],
                           pts_hbm.at[b, pl.ds(lo * 2, _QI * 2)], sem),
          pltpu.async_copy(pts_v.at[pl.ds(2 * NIMP + q * _QC * 2, _QC * 2)],
                           pts_hbm.at[b, pl.ds(2 * NIMP + q * _QC * 2, _QC * 2)], sem)]
    for (src, dst) in ((gidx_v, gidx_hbm), (gw_v, gw_hbm), (cn_v, cn_hbm)):
        ho.append(pltpu.async_copy(src.at[pl.ds(lo * 4, _QI * 4)],
                                   dst.at[b, pl.ds(lo * 4, _QI * 4)], sem))
        ho.append(pltpu.async_copy(src.at[pl.ds(4 * NIMP + q * _QC * 4, _QC * 4)],
                                   dst.at[b, pl.ds(4 * NIMP + q * _QC * 4, _QC * 4)], sem))
    for h in ho:
        h.wait()


def _k_sel(rank, outf, overt, covt):
    return pl.kernel(
        _sel_body,
        out_type=(
            jax.ShapeDtypeStruct((B, 2 * NPT), jnp.float32),
            jax.ShapeDtypeStruct((B, 4 * NPT), jnp.int32),
            jax.ShapeDtypeStruct((B, 4 * NPT), jnp.float32),
            jax.ShapeDtypeStruct((B, 4 * NPT), jnp.float32),
        ),
        mesh=_mesh,
        compiler_params=pltpu.CompilerParams(needs_layout_passes=False),
        scratch_types=[
            pltpu.VMEM((KN,), jnp.int32),
            pltpu.VMEM((KN,), jnp.float32),
            pltpu.VMEM((KN,), jnp.float32),
            pltpu.VMEM((_QC,), jnp.float32),
            pltpu.VMEM((_QC,), jnp.float32),
            pltpu.VMEM((3 * 1024,), jnp.float32),
            pltpu.VMEM((_QI,), jnp.int32),
            pltpu.VMEM((2 * NPT,), jnp.float32),
            pltpu.VMEM((4 * NPT,), jnp.int32),
            pltpu.VMEM((4 * NPT,), jnp.float32),
            pltpu.VMEM((4 * NPT,), jnp.float32),
            pltpu.SemaphoreType.DMA,
        ],
    )(rank, outf, overt, covt)


_FCP = 256
_CH = 128
_CPT = NPT // 4


def _fine_body(res2t_hbm, gidx_hbm, gw_hbm, fine_hbm,
               idx0, idx1, w0, w1, rows0, rows1, out_v, sem0, sem1):
    wid = _wid()
    b = wid // 4
    q = wid % 4
    base = q * (4 * _CPT)
    nchunks = (4 * _CPT) // _CH
    ppc = _CH // 4
    idxs = (idx0, idx1); wv = (w0, w1); rows = (rows0, rows1); sems = (sem0, sem1)

    def start(k):
        s = k % 2
        pltpu.sync_copy(gidx_hbm.at[b, pl.ds(base + k * _CH, _CH)], idxs[s])
        pltpu.sync_copy(gw_hbm.at[b, pl.ds(base + k * _CH, _CH)], wv[s])
        return pltpu.async_copy(res2t_hbm.at[idxs[s]], rows[s], sems[s])

    handles = [start(0), None]
    for k in range(nchunks):
        s = k % 2
        handles[s].wait()
        if k + 1 < nchunks:
            handles[1 - s] = start(k + 1)
        rows_v = rows[s]
        w_v = wv[s]

        def point(p, _):
            rb = p * 4
            ws = [plsc.load_gather(w_v, [jnp.full((L,), rb + j, jnp.int32)])
                  for j in range(4)]
            for c in range(FC // L):
                acc = None
                for j in range(4):
                    t = ws[j] * rows_v[rb + j, pl.ds(c * L, L)]
                    acc = t if acc is None else acc + t
                out_v[p, pl.ds(c * L, L)] = acc
            return _
        lax.fori_loop(0, ppc, point, None)
        pltpu.sync_copy(out_v, fine_hbm.at[b, pl.ds(q * _CPT + k * ppc, ppc), :])


def _k_fine(res2t, gidx, gw):
    return pl.kernel(
        _fine_body,
        out_type=jax.ShapeDtypeStruct((B, NPT, FC), jnp.float32),
        mesh=_mesh,
        compiler_params=pltpu.CompilerParams(needs_layout_passes=False),
        scratch_types=[
            pltpu.VMEM((_CH,), jnp.int32),
            pltpu.VMEM((_CH,), jnp.int32),
            pltpu.VMEM((_CH,), jnp.float32),
            pltpu.VMEM((_CH,), jnp.float32),
            pltpu.VMEM((_CH, _FCP), jnp.float32),
            pltpu.VMEM((_CH, _FCP), jnp.float32),
            pltpu.VMEM((_CH // 4, FC), jnp.float32),
            pltpu.SemaphoreType.DMA,
            pltpu.SemaphoreType.DMA,
        ],
    )(res2t, gidx, gw)


def _mlp_body(fine_ref, cn_ref, w1f_ref, w1c_ref, b1_ref, w2_ref, b2_ref,
              w3_ref, b3_ref, rend_ref):
    f = fine_ref[0]
    cn = cn_ref[0]
    h1 = jnp.dot(f, w1f_ref[...], preferred_element_type=jnp.float32)
    h1 = h1 + jnp.dot(cn, w1c_ref[...], preferred_element_type=jnp.float32)
    h1 = jnp.maximum(h1 + b1_ref[...], 0.0)
    h2 = jnp.dot(h1, w2_ref[...], preferred_element_type=jnp.float32)
    h2 = jnp.maximum(h2 + b2_ref[...], 0.0)
    r = lax.dot_general(w3_ref[...], h2, (((1,), (1,)), ((), ())),
                        preferred_element_type=jnp.float32)
    rend_ref[0] = r + b3_ref[...]


def _k_mlp(fine, cn, w1f, w1c, b1, w2, b2, w3, b3):
    return pl.pallas_call(
        _mlp_body,
        grid=(B,),
        in_specs=[
            pl.BlockSpec((1, NPT, FC), lambda b: (b, 0, 0)),
            pl.BlockSpec((1, NPT, 4), lambda b: (b, 0, 0)),
            pl.BlockSpec((FC, 128), lambda b: (0, 0)),
            pl.BlockSpec((4, 128), lambda b: (0, 0)),
            pl.BlockSpec((1, 128), lambda b: (0, 0)),
            pl.BlockSpec((128, 128), lambda b: (0, 0)),
            pl.BlockSpec((1, 128), lambda b: (0, 0)),
            pl.BlockSpec((3, 128), lambda b: (0, 0)),
            pl.BlockSpec((3, 1), lambda b: (0, 0)),
        ],
        out_specs=pl.BlockSpec((1, 3, NPT), lambda b: (b, 0, 0)),
        out_shape=jax.ShapeDtypeStruct((B, 3, NPT), jnp.float32),
    )(fine, cn, w1f, w1c, b1, w2, b2, w3, b3)


def kernel(x, res2, out, W1, b1, W2, b2, W3, b3):
    outf = out.reshape(B, 3 * 1024)
    unc = _k_unc(outf, jnp.asarray(_OIDX_T), jnp.asarray(_OW_T))
    rank = _k_rank(unc, unc.reshape(B, KN, 1))
    pts_f, gidx, gw, cn = _k_sel(rank, outf,
                                 jnp.asarray(_OVER_T), jnp.asarray(_COV_T))
    res2t = jnp.pad(jnp.transpose(res2, (0, 2, 3, 1)),
                    ((0, 0), (0, 0), (0, 0), (0, _FCP - FC))
                    ).reshape(B * FS * FS, _FCP)
    fine = _k_fine(res2t, gidx, gw)
    w1f = W1[:, :FC].T
    w1c = jnp.pad(W1[:, FC:], ((0, 0), (0, 1))).T
    rend = _k_mlp(fine, cn.reshape(B, NPT, 4), w1f, w1c, b1[None, :],
                  W2.T, b2[None, :], W3, b3[:, None])
    points = pts_f.reshape(B, NPT, 2)
    return (out, rend, points)
